# baseline (speedup 1.0000x reference)
import numpy as np
import jax
import jax.numpy as jnp
from jax import lax
from jax.experimental import pallas as pl
from jax.experimental.pallas import tpu as pltpu
from jax.experimental.pallas import tpu_sc as plsc

N = 10000
E = 320000
D_IN = 128
D_H = 64
D_OUT = 32

NC = 2
NS = 16
NW = NC * NS

NPAD = 10240
CH = 125
KJ = 80
EPW = KJ * CH
NBUF = 4
NG = KJ // NBUF
ROWS_PT = NPAD // NS
ROWS_PW = NPAD // NW

_mesh = plsc.VectorSubcoreMesh(
    core_axis_name="c", subcore_axis_name="s", num_cores=NC, num_subcores=NS)


def _deg_body(dstv_hbm, zeros16_hbm, ones_hbm, out_hbm, dst_v, ones_v, deg_sh):
    c = lax.axis_index("c")
    s = lax.axis_index("s")
    w = c * NS + s
    rs = s * ROWS_PT
    pltpu.sync_copy(zeros16_hbm.at[pl.ds(rs, ROWS_PT)], deg_sh.at[pl.ds(rs, ROWS_PT)])
    pltpu.sync_copy(ones_hbm, ones_v)
    pltpu.sync_copy(dstv_hbm.at[w], dst_v)
    plsc.subcore_barrier()

    def body(g, carry):
        pltpu.sync_copy(ones_v, deg_sh.at[dst_v.at[g]], add=True)
        return carry

    lax.fori_loop(0, KJ, body, 0)
    plsc.subcore_barrier()
    pltpu.sync_copy(deg_sh.at[pl.ds(rs, ROWS_PT)], out_hbm.at[c, pl.ds(rs, ROWS_PT)])


def _rsqrt16(d):
    i = plsc.bitcast(d, jnp.int32)
    i = jnp.int32(0x5F3759DF) - lax.shift_right_arithmetic(i, 1)
    y = plsc.bitcast(i, jnp.float32)
    for _ in range(3):
        y = y * (1.5 - 0.5 * d * y * y)
    return y


def _scale1_body(h0_hbm, degp_hbm, out_hbm, fbuf, dbuf0, dbuf1):
    c = lax.axis_index("c")
    s = lax.axis_index("s")
    r0 = (c * NS + s) * ROWS_PW
    pltpu.sync_copy(h0_hbm.at[pl.ds(r0, ROWS_PW)], fbuf)
    pltpu.sync_copy(degp_hbm.at[0, pl.ds(r0, ROWS_PW)], dbuf0)
    pltpu.sync_copy(degp_hbm.at[1, pl.ds(r0, ROWS_PW)], dbuf1)

    def rowbody(i, carry):
        for u in range(4):
            r = 4 * i + u
            y = _rsqrt16(dbuf0[r, :] + dbuf1[r, :] + 1.0)
            for k in range(D_H // 16):
                sl = pl.ds(16 * k, 16)
                fbuf[r, sl] = fbuf[r, sl] * y
        return carry

    lax.fori_loop(0, ROWS_PW // 4, rowbody, 0)
    pltpu.sync_copy(fbuf, out_hbm.at[pl.ds(r0, ROWS_PW)])


def _mid2_body(acc1_hbm, feat1_hbm, degp_hbm, b1_hbm, out_hbm,
               fbuf, ab0, ab1, dbuf0, dbuf1, b1_v):
    c = lax.axis_index("c")
    s = lax.axis_index("s")
    r0 = (c * NS + s) * ROWS_PW
    pltpu.sync_copy(feat1_hbm.at[pl.ds(r0, ROWS_PW)], fbuf)
    pltpu.sync_copy(acc1_hbm.at[0, pl.ds(r0, ROWS_PW)], ab0)
    pltpu.sync_copy(acc1_hbm.at[1, pl.ds(r0, ROWS_PW)], ab1)
    pltpu.sync_copy(degp_hbm.at[0, pl.ds(r0, ROWS_PW)], dbuf0)
    pltpu.sync_copy(degp_hbm.at[1, pl.ds(r0, ROWS_PW)], dbuf1)
    pltpu.sync_copy(b1_hbm, b1_v)

    def rowbody(i, carry):
        for u in range(4):
            r = 4 * i + u
            y = _rsqrt16(dbuf0[r, :] + dbuf1[r, :] + 1.0)
            for k in range(D_H // 16):
                sl = pl.ds(16 * k, 16)
                p = (ab0[r, sl] + ab1[r, sl] + fbuf[r, sl]) * y
                hv = jnp.maximum(p + b1_v[sl], 0.0)
                fbuf[r, sl] = hv * y
        return carry

    lax.fori_loop(0, ROWS_PW // 4, rowbody, 0)
    pltpu.sync_copy(fbuf, out_hbm.at[pl.ds(r0, ROWS_PW)])


def _prop_body(feat_hbm, srcv_hbm, dstv_hbm, zeros_hbm, out_hbm,
               src_v, dst_v, rows_a, rows_b, acc_sh, sem_a, sem_b):
    c = lax.axis_index("c")
    s = lax.axis_index("s")
    w = c * NS + s
    rs = s * ROWS_PT
    pltpu.sync_copy(zeros_hbm.at[pl.ds(rs, ROWS_PT)], acc_sh.at[pl.ds(rs, ROWS_PT)])
    pltpu.sync_copy(srcv_hbm.at[w], src_v)
    pltpu.sync_copy(dstv_hbm.at[w], dst_v)
    plsc.subcore_barrier()

    def fire(g, rows, sem):
        for b in range(NBUF):
            pltpu.async_copy(feat_hbm.at[src_v.at[g * NBUF + b]], rows.at[b], sem)

    def drain_scatter(g, rows, sem):
        for b in range(NBUF):
            pltpu.make_async_copy(
                feat_hbm.at[src_v.at[g * NBUF + b]], rows.at[b], sem).wait()
        for b in range(NBUF):
            pltpu.sync_copy(rows.at[b], acc_sh.at[dst_v.at[g * NBUF + b]], add=True)

    fire(0, rows_a, sem_a)

    def body(k, carry):
        g0 = 2 * k
        fire(g0 + 1, rows_b, sem_b)
        drain_scatter(g0, rows_a, sem_a)
        fire(g0 + 2, rows_a, sem_a)
        drain_scatter(g0 + 1, rows_b, sem_b)
        return carry

    lax.fori_loop(0, NG // 2 - 1, body, 0)
    g0 = NG - 2
    fire(g0 + 1, rows_b, sem_b)
    drain_scatter(g0, rows_a, sem_a)
    drain_scatter(g0 + 1, rows_b, sem_b)
    plsc.subcore_barrier()
    pltpu.sync_copy(acc_sh.at[pl.ds(rs, ROWS_PT)], out_hbm.at[c, pl.ds(rs, ROWS_PT)])


_deg_kernel = pl.kernel(
    _deg_body,
    out_type=jax.ShapeDtypeStruct((NC, NPAD, 16), jnp.float32),
    mesh=_mesh,
    scratch_types=[
        pltpu.VMEM((KJ, CH), jnp.int32),
        pltpu.VMEM((CH, 16), jnp.float32),
        pltpu.VMEM_SHARED((NPAD, 16), jnp.float32),
    ],
    compiler_params=pltpu.CompilerParams(use_tc_tiling_on_sc=False),
)

_scale1_kernel = pl.kernel(
    _scale1_body,
    out_type=jax.ShapeDtypeStruct((NPAD, D_H), jnp.float32),
    mesh=_mesh,
    scratch_types=[
        pltpu.VMEM((ROWS_PW, D_H), jnp.float32),
        pltpu.VMEM((ROWS_PW, 16), jnp.float32),
        pltpu.VMEM((ROWS_PW, 16), jnp.float32),
    ],
    compiler_params=pltpu.CompilerParams(
        use_tc_tiling_on_sc=False, needs_layout_passes=False),
)

_mid2_kernel = pl.kernel(
    _mid2_body,
    out_type=jax.ShapeDtypeStruct((NPAD, D_H), jnp.float32),
    mesh=_mesh,
    scratch_types=[
        pltpu.VMEM((ROWS_PW, D_H), jnp.float32),
        pltpu.VMEM((ROWS_PW, D_H), jnp.float32),
        pltpu.VMEM((ROWS_PW, D_H), jnp.float32),
        pltpu.VMEM((ROWS_PW, 16), jnp.float32),
        pltpu.VMEM((ROWS_PW, 16), jnp.float32),
        pltpu.VMEM((D_H,), jnp.float32),
    ],
    compiler_params=pltpu.CompilerParams(
        use_tc_tiling_on_sc=False, needs_layout_passes=False),
)

_prop_kernel = pl.kernel(
    _prop_body,
    out_type=jax.ShapeDtypeStruct((NC, NPAD, D_H), jnp.float32),
    mesh=_mesh,
    scratch_types=[
        pltpu.VMEM((KJ, CH), jnp.int32),
        pltpu.VMEM((KJ, CH), jnp.int32),
        pltpu.VMEM((NBUF, CH, D_H), jnp.float32),
        pltpu.VMEM((NBUF, CH, D_H), jnp.float32),
        pltpu.VMEM_SHARED((NPAD, D_H), jnp.float32),
        pltpu.SemaphoreType.DMA,
        pltpu.SemaphoreType.DMA,
    ],
    compiler_params=pltpu.CompilerParams(use_tc_tiling_on_sc=False),
)


_BLK = 1024
_OBLK = 1000


def _pre_body(x_ref, w1_ref, o_ref):
    o_ref[...] = jnp.dot(x_ref[...], w1_ref[...], preferred_element_type=jnp.float32)


def _out_body(acc_ref, feat_ref, deg_ref,
              wmu_ref, bmu_ref, wls_ref, bls_ref, mu_ref, ls_ref):
    dinv = lax.rsqrt(deg_ref[0, :, :1] + deg_ref[1, :, :1] + 1.0)
    p = (acc_ref[0] + acc_ref[1] + feat_ref[...]) * dinv
    mu_ref[...] = jnp.dot(p, wmu_ref[...], preferred_element_type=jnp.float32) + bmu_ref[...]
    ls_ref[...] = jnp.dot(p, wls_ref[...], preferred_element_type=jnp.float32) + bls_ref[...]


def _tc_pre(x, w1):
    return pl.pallas_call(
        _pre_body,
        grid=(NPAD // _BLK,),
        in_specs=[pl.BlockSpec((_BLK, D_IN), lambda i: (i, 0)),
                  pl.BlockSpec((D_IN, D_H), lambda i: (0, 0))],
        out_specs=pl.BlockSpec((_BLK, D_H), lambda i: (i, 0)),
        out_shape=jax.ShapeDtypeStruct((NPAD, D_H), jnp.float32),
    )(x, w1)


def _tc_out(acc, feat, degp, wmu, bmu, wls, bls):
    return pl.pallas_call(
        _out_body,
        grid=(N // _OBLK,),
        in_specs=[pl.BlockSpec((NC, _OBLK, D_H), lambda i: (0, i, 0)),
                  pl.BlockSpec((_OBLK, D_H), lambda i: (i, 0)),
                  pl.BlockSpec((NC, _OBLK, 16), lambda i: (0, i, 0)),
                  pl.BlockSpec((D_H, D_OUT), lambda i: (0, 0)),
                  pl.BlockSpec((1, D_OUT), lambda i: (0, 0)),
                  pl.BlockSpec((D_H, D_OUT), lambda i: (0, 0)),
                  pl.BlockSpec((1, D_OUT), lambda i: (0, 0))],
        out_specs=[pl.BlockSpec((_OBLK, D_OUT), lambda i: (i, 0)),
                   pl.BlockSpec((_OBLK, D_OUT), lambda i: (i, 0))],
        out_shape=[jax.ShapeDtypeStruct((N, D_OUT), jnp.float32),
                   jax.ShapeDtypeStruct((N, D_OUT), jnp.float32)],
    )(acc, feat, degp, wmu, bmu, wls, bls)


def kernel(x, edge_index, W1, b1, W_mu, b_mu, W_ls, b_ls):
    srcv = edge_index[0].reshape(NW, KJ, CH)
    dstv = edge_index[1].reshape(NW, KJ, CH)

    xp = jnp.pad(x, ((0, NPAD - N), (0, 0)))
    zeros64 = jnp.zeros((NPAD, D_H), jnp.float32)
    zeros16 = jnp.zeros((NPAD, 16), jnp.float32)
    ones = jnp.ones((CH, 16), jnp.float32)

    degp = _deg_kernel(dstv, zeros16, ones)
    h0 = _tc_pre(xp, W1)
    feat1 = _scale1_kernel(h0, degp)
    acc1 = _prop_kernel(feat1, srcv, dstv, zeros64)
    feat2 = _mid2_kernel(acc1, feat1, degp, b1)
    acc2 = _prop_kernel(feat2, srcv, dstv, zeros64)
    mu, ls = _tc_out(acc2, feat2, degp,
                     W_mu, b_mu.reshape(1, D_OUT), W_ls, b_ls.reshape(1, D_OUT))
    return (mu, ls)
